# Initial kernel scaffold; baseline (speedup 1.0000x reference)
import jax
import jax.numpy as jnp
from jax.experimental import pallas as pl


def kernel(x, thresholds):
    raise NotImplementedError("write your pallas kernel here")


# speedup vs baseline: 75.3576x; 75.3576x over previous
import functools

import jax
import jax.numpy as jnp
from jax import lax
from jax.experimental import pallas as pl
from jax.experimental.pallas import tpu as pltpu
from jax.experimental.pallas import tpu_sc as plsc

_NC = 2
_NS = 16
_L = 16
_NW = _NC * _NS


def _math_gcd(a, b):
    while b:
        a, b = b, a % b
    return a


@functools.partial(jax.jit, static_argnums=(4, 5, 6))
def _run(x_flat, thr_s, table, ftab, chunk, period, f):
    n = x_flat.shape[0]
    t1 = thr_s.shape[0] // f
    pw = period * _L

    @functools.partial(
        pl.kernel,
        out_type=jax.ShapeDtypeStruct((n,), jnp.float32),
        mesh=plsc.VectorSubcoreMesh(core_axis_name="c", subcore_axis_name="s"),
        compiler_params=pltpu.CompilerParams(needs_layout_passes=False),
        scratch_types=[
            pltpu.VMEM((chunk,), jnp.float32),
            pltpu.VMEM((f * t1,), jnp.float32),
            pltpu.VMEM((f * t1,), jnp.float32),
            pltpu.VMEM((pw,), jnp.int32),
        ],
    )
    def _sc(x_hbm, thr_hbm, tab_hbm, ftab_hbm, out_hbm, buf, thr_v, tab_v, ftab_v):
        wid = lax.axis_index("s") * _NC + lax.axis_index("c")
        base = wid * chunk
        pltpu.sync_copy(x_hbm.at[pl.ds(base, chunk)], buf)
        pltpu.sync_copy(thr_hbm, thr_v)
        pltpu.sync_copy(tab_hbm, tab_v)
        pltpu.sync_copy(ftab_hbm, ftab_v)

        def outer(k, carry):
            kb = k * pw
            for j in range(period):
                off = kb + j * _L
                xv = buf[pl.ds(off, _L)]
                fb = ftab_v[pl.ds(j * _L, _L)]
                idx = jnp.zeros((_L,), jnp.int32)
                for s in (8, 4, 2, 1):
                    cand = idx + s
                    tv = plsc.load_gather(thr_v, [fb + cand])
                    idx = jnp.where(xv > tv, cand, idx)
                buf[pl.ds(off, _L)] = plsc.load_gather(tab_v, [fb + idx])
            return carry

        lax.fori_loop(0, chunk // pw, outer, 0)
        pltpu.sync_copy(buf, out_hbm.at[pl.ds(base, chunk)])

    return _sc(x_flat, thr_s, table, ftab)


def kernel(x, thresholds):
    b, f = x.shape
    t = thresholds.shape[1]
    assert t == 15, "binary-search schedule is built for 15 thresholds"

    d = jnp.diff(thresholds, axis=1)
    d = jnp.concatenate([-d[:, :1], d, d[:, -1:]], axis=1)
    thr_cat = jnp.concatenate([thresholds[:, :1], thresholds], axis=1)
    table = thr_cat + d * 0.5
    thr_s = thr_cat

    total = b * f
    assert total % _NW == 0
    chunk = total // _NW
    period = f // _math_gcd(_L, f)
    assert chunk % (period * _L) == 0
    t1 = t + 1
    ftab = (jnp.arange(period * _L, dtype=jnp.int32) % f) * t1

    out = _run(x.reshape(-1), thr_s.reshape(-1), table.reshape(-1), ftab,
               chunk, period, f)
    return out.reshape(b, f)


# speedup vs baseline: 154.3793x; 2.0486x over previous
import functools

import jax
import jax.numpy as jnp
from jax import lax
from jax.experimental import pallas as pl
from jax.experimental.pallas import tpu as pltpu
from jax.experimental.pallas import tpu_sc as plsc

_NC = 2
_NS = 16
_L = 16
_NW = _NC * _NS


def _math_gcd(a, b):
    while b:
        a, b = b, a % b
    return a


@functools.partial(jax.jit, static_argnums=(4, 5, 6))
def _run(x_flat, thr_s, table, ftab, chunk, period, f):
    n = x_flat.shape[0]
    t1 = thr_s.shape[0] // f
    pw = period * _L

    @functools.partial(
        pl.kernel,
        out_type=jax.ShapeDtypeStruct((n,), jnp.float32),
        mesh=plsc.VectorSubcoreMesh(core_axis_name="c", subcore_axis_name="s"),
        compiler_params=pltpu.CompilerParams(needs_layout_passes=False),
        scratch_types=[
            pltpu.VMEM((chunk,), jnp.float32),
            pltpu.VMEM((chunk,), jnp.float32),
            pltpu.VMEM((f * t1,), jnp.float32),
            pltpu.VMEM((f * t1,), jnp.float32),
            pltpu.VMEM((pw,), jnp.int32),
        ],
    )
    def _sc(x_hbm, thr_hbm, tab_hbm, ftab_hbm, out_hbm,
            buf_in, buf_out, thr_v, tab_v, ftab_v):
        wid = lax.axis_index("s") * _NC + lax.axis_index("c")
        base = wid * chunk
        pltpu.sync_copy(x_hbm.at[pl.ds(base, chunk)], buf_in)
        pltpu.sync_copy(thr_hbm, thr_v)
        pltpu.sync_copy(tab_hbm, tab_v)
        pltpu.sync_copy(ftab_hbm, ftab_v)

        @plsc.parallel_loop(0, chunk, step=pw)
        def _block(kb):
            for j in range(period):
                off = kb + j * _L
                xv = buf_in[pl.ds(off, _L)]
                fb = ftab_v[pl.ds(j * _L, _L)]
                idx = jnp.zeros((_L,), jnp.int32)
                for s in (8, 4, 2, 1):
                    cand = idx + s
                    tv = plsc.load_gather(thr_v, [fb + cand])
                    idx = jnp.where(xv > tv, cand, idx)
                buf_out[pl.ds(off, _L)] = plsc.load_gather(tab_v, [fb + idx])

        pltpu.sync_copy(buf_out, out_hbm.at[pl.ds(base, chunk)])

    return _sc(x_flat, thr_s, table, ftab)


def kernel(x, thresholds):
    b, f = x.shape
    t = thresholds.shape[1]
    assert t == 15, "binary-search schedule is built for 15 thresholds"

    d = jnp.diff(thresholds, axis=1)
    d = jnp.concatenate([-d[:, :1], d, d[:, -1:]], axis=1)
    thr_cat = jnp.concatenate([thresholds[:, :1], thresholds], axis=1)
    table = thr_cat + d * 0.5
    thr_s = thr_cat

    total = b * f
    assert total % _NW == 0
    chunk = total // _NW
    period = f // _math_gcd(_L, f)
    assert chunk % (period * _L) == 0
    t1 = t + 1
    ftab = (jnp.arange(period * _L, dtype=jnp.int32) % f) * t1

    out = _run(x.reshape(-1), thr_s.reshape(-1), table.reshape(-1), ftab,
               chunk, period, f)
    return out.reshape(b, f)
